# Initial kernel scaffold; baseline (speedup 1.0000x reference)
import jax
import jax.numpy as jnp
from jax.experimental import pallas as pl


def kernel(semantic_state, token_ids):
    raise NotImplementedError("write your pallas kernel here")


# speedup vs baseline: 1.2615x; 1.2615x over previous
import functools

import jax
import jax.numpy as jnp
from jax import lax
from jax.experimental import pallas as pl
from jax.experimental.pallas import tpu as pltpu
from jax.experimental.pallas import tpu_sc as plsc

L = 4096
D = 64
NUM_BLOCKS = 16
BLOCK_DIM = 4
V = 512
NTILES = 32
ROWS_PER_TILE = L // NTILES
CW = 16


def _norm_body(x_ref, tbn_ref, q_ref):
    x = x_ref[...]
    x2 = x * x
    lane = lax.broadcasted_iota(jnp.int32, (D, NUM_BLOCKS), 0)
    blk = lax.broadcasted_iota(jnp.int32, (D, NUM_BLOCKS), 1)
    g = (lane // BLOCK_DIM == blk).astype(jnp.float32)
    ss = lax.dot_general(x2, g, (((1,), (0,)), ((), ())),
                         precision=lax.Precision.HIGHEST)
    nrm = jnp.maximum(jnp.sqrt(ss), 1e-12)
    nexp = lax.dot_general(nrm, g.T, (((1,), (0,)), ((), ())),
                           precision=lax.Precision.HIGHEST)
    tbn = x / nexp
    tbn_ref[...] = tbn
    q_ref[...] = jnp.sum(tbn * tbn).reshape(1, 1)


def _finish_body(acc_ref, cnt_ref, q_ref, out_ref):
    s = acc_ref[0] + acc_ref[1]
    ssum = jnp.sum(s * s)
    c = cnt_ref[0] + cnt_ref[1]
    pairs = jnp.sum(c * c - c) / (2.0 * CW)
    q = jnp.sum(q_ref[...])
    total = (ssum - q) / (2.0 * NUM_BLOCKS)
    out_ref[...] = jnp.where(pairs > 0.5, total / pairs, 0.0).reshape(1, 1)


def _sc_scatter_body(tbn_hbm, tok_hbm, zacc_hbm, zcnt_hbm, ones_hbm,
                     out_acc, out_cnt,
                     rows_v, idx_v, ones_v, acc_sh, cnt_sh):
    cid = lax.axis_index("c")
    sid = lax.axis_index("s")
    base = (cid * 16 + sid) * ROWS_PER_TILE
    pltpu.sync_copy(tok_hbm.at[pl.ds(base, ROWS_PER_TILE)], idx_v)
    pltpu.sync_copy(tbn_hbm.at[pl.ds(base, ROWS_PER_TILE)], rows_v)
    pltpu.sync_copy(ones_hbm, ones_v)

    @pl.when(sid == 0)
    def _init():
        pltpu.sync_copy(zacc_hbm, acc_sh)
        pltpu.sync_copy(zcnt_hbm, cnt_sh)

    plsc.subcore_barrier()
    pltpu.sync_copy(rows_v, acc_sh.at[idx_v], add=True)
    pltpu.sync_copy(ones_v, cnt_sh.at[idx_v], add=True)
    plsc.subcore_barrier()

    @pl.when(sid == 0)
    def _flush():
        pltpu.sync_copy(acc_sh, out_acc.at[cid])
        pltpu.sync_copy(cnt_sh, out_cnt.at[cid])


_sc_scatter = functools.partial(
    pl.kernel,
    out_type=[
        jax.ShapeDtypeStruct((2, V, D), jnp.float32),
        jax.ShapeDtypeStruct((2, V, CW), jnp.float32),
    ],
    mesh=plsc.VectorSubcoreMesh(core_axis_name="c", subcore_axis_name="s"),
    scratch_types=[
        pltpu.VMEM((ROWS_PER_TILE, D), jnp.float32),
        pltpu.VMEM((ROWS_PER_TILE,), jnp.int32),
        pltpu.VMEM((ROWS_PER_TILE, CW), jnp.float32),
        pltpu.VMEM_SHARED((V, D), jnp.float32),
        pltpu.VMEM_SHARED((V, CW), jnp.float32),
    ],
)(_sc_scatter_body)


def kernel(semantic_state, token_ids):
    x = semantic_state.reshape(L, D)
    tok = token_ids.reshape(L)

    tbn, q = pl.pallas_call(
        _norm_body,
        out_shape=[
            jax.ShapeDtypeStruct((L, D), jnp.float32),
            jax.ShapeDtypeStruct((1, 1), jnp.float32),
        ],
    )(x)

    zacc = jnp.zeros((V, D), jnp.float32)
    zcnt = jnp.zeros((V, CW), jnp.float32)
    ones = jnp.ones((ROWS_PER_TILE, CW), jnp.float32)
    acc_p, cnt_p = _sc_scatter(tbn, tok, zacc, zcnt, ones)

    loss = pl.pallas_call(
        _finish_body,
        out_shape=jax.ShapeDtypeStruct((1, 1), jnp.float32),
    )(acc_p, cnt_p, q)
    return loss.reshape(())


# speedup vs baseline: 1.3423x; 1.0641x over previous
import functools

import jax
import jax.numpy as jnp
from jax import lax
from jax.experimental import pallas as pl
from jax.experimental.pallas import tpu as pltpu
from jax.experimental.pallas import tpu_sc as plsc

L = 4096
D = 64
NUM_BLOCKS = 16
BLOCK_DIM = 4
V = 512
NTILES = 32
ROWS_PER_TILE = L // NTILES
CW = 16
VS = V // 16


def _norm_body(x_ref, tbn_ref, q_ref):
    x = x_ref[...]
    x2 = x * x
    lane = lax.broadcasted_iota(jnp.int32, (D, NUM_BLOCKS), 0)
    blk = lax.broadcasted_iota(jnp.int32, (D, NUM_BLOCKS), 1)
    g = (lane // BLOCK_DIM == blk).astype(jnp.float32)
    ss = lax.dot_general(x2, g, (((1,), (0,)), ((), ())),
                         precision=lax.Precision.HIGHEST)
    nrm = jnp.maximum(jnp.sqrt(ss), 1e-12)
    inv = 1.0 / nrm
    invexp = lax.dot_general(inv, g.T, (((1,), (0,)), ((), ())),
                             precision=lax.Precision.HIGHEST)
    tbn_ref[...] = x * invexp
    q_ref[...] = jnp.sum(ss * inv * inv).reshape(1, 1)


def _finish_body(acc_ref, cnt_ref, q_ref, out_ref):
    s = acc_ref[0] + acc_ref[1]
    ssum = jnp.sum(s * s)
    c = cnt_ref[0] + cnt_ref[1]
    pairs = jnp.sum(c * c - c) / (2.0 * CW)
    q = jnp.sum(q_ref[...])
    total = (ssum - q) / (2.0 * NUM_BLOCKS)
    out_ref[...] = jnp.where(pairs > 0.5, total / pairs, 0.0).reshape(1, 1)


def _sc_scatter_body(tbn_hbm, tok_hbm, zacc_hbm, zcnt_hbm, ones_hbm,
                     out_acc, out_cnt,
                     rows_v, idx_v, ones_v, acc_sh, cnt_sh, sems):
    cid = lax.axis_index("c")
    sid = lax.axis_index("s")
    base = (cid * 16 + sid) * ROWS_PER_TILE
    vb = sid * VS
    cps = [
        pltpu.async_copy(tok_hbm.at[pl.ds(base, ROWS_PER_TILE)], idx_v,
                         sems.at[0]),
        pltpu.async_copy(tbn_hbm.at[pl.ds(base, ROWS_PER_TILE)], rows_v,
                         sems.at[1]),
        pltpu.async_copy(ones_hbm, ones_v, sems.at[2]),
        pltpu.async_copy(zacc_hbm.at[pl.ds(vb, VS)], acc_sh.at[pl.ds(vb, VS)],
                         sems.at[3]),
        pltpu.async_copy(zcnt_hbm.at[pl.ds(vb, VS)], cnt_sh.at[pl.ds(vb, VS)],
                         sems.at[4]),
    ]
    for cp in cps:
        cp.wait()
    plsc.subcore_barrier()
    pltpu.sync_copy(rows_v, acc_sh.at[idx_v], add=True)
    pltpu.sync_copy(ones_v, cnt_sh.at[idx_v], add=True)
    plsc.subcore_barrier()
    fps = [
        pltpu.async_copy(acc_sh.at[pl.ds(vb, VS)],
                         out_acc.at[cid, pl.ds(vb, VS)], sems.at[0]),
        pltpu.async_copy(cnt_sh.at[pl.ds(vb, VS)],
                         out_cnt.at[cid, pl.ds(vb, VS)], sems.at[1]),
    ]
    for cp in fps:
        cp.wait()


_sc_scatter = functools.partial(
    pl.kernel,
    out_type=[
        jax.ShapeDtypeStruct((2, V, D), jnp.float32),
        jax.ShapeDtypeStruct((2, V, CW), jnp.float32),
    ],
    mesh=plsc.VectorSubcoreMesh(core_axis_name="c", subcore_axis_name="s"),
    scratch_types=[
        pltpu.VMEM((ROWS_PER_TILE, D), jnp.float32),
        pltpu.VMEM((ROWS_PER_TILE,), jnp.int32),
        pltpu.VMEM((ROWS_PER_TILE, CW), jnp.float32),
        pltpu.VMEM_SHARED((V, D), jnp.float32),
        pltpu.VMEM_SHARED((V, CW), jnp.float32),
        pltpu.SemaphoreType.DMA((5,)),
    ],
)(_sc_scatter_body)


def kernel(semantic_state, token_ids):
    x = semantic_state.reshape(L, D)
    tok = token_ids.reshape(L)

    tbn, q = pl.pallas_call(
        _norm_body,
        out_shape=[
            jax.ShapeDtypeStruct((L, D), jnp.float32),
            jax.ShapeDtypeStruct((1, 1), jnp.float32),
        ],
    )(x)

    zacc = jnp.zeros((V, D), jnp.float32)
    zcnt = jnp.zeros((V, CW), jnp.float32)
    ones = jnp.ones((ROWS_PER_TILE, CW), jnp.float32)
    acc_p, cnt_p = _sc_scatter(tbn, tok, zacc, zcnt, ones)

    loss = pl.pallas_call(
        _finish_body,
        out_shape=jax.ShapeDtypeStruct((1, 1), jnp.float32),
    )(acc_p, cnt_p, q)
    return loss.reshape(())
